# baseline (speedup 1.0000x reference)
import jax
import jax.numpy as jnp
from jax.experimental import pallas as pl


def _mlp_block(x_ref, gua_ref, gub_ref, dpa_ref, dpb_ref, o_ref):
    ih = dpa_ref.shape[1]
    xb = x_ref[...].astype(jnp.bfloat16)
    gate = jnp.dot(xb, gua_ref[0].astype(jnp.bfloat16),
                   preferred_element_type=jnp.float32)
    up = jnp.dot(xb, gub_ref[0].astype(jnp.bfloat16),
                 preferred_element_type=jnp.float32)
    inter = (jax.nn.silu(gate) * up).astype(jnp.bfloat16)
    o_ref[...] = (
        jnp.dot(inter[:, :ih], dpa_ref[0].astype(jnp.bfloat16),
                preferred_element_type=jnp.float32)
        + jnp.dot(inter[:, ih:], dpb_ref[0].astype(jnp.bfloat16),
                  preferred_element_type=jnp.float32)
    )


def kernel(x, gate_up_proj, down_proj):
    b, n, h = x.shape
    e, _, i2 = gate_up_proj.shape
    i = i2 // 2
    g = n // e
    rows = b * g
    x3 = x.reshape(rows, e * h)
    out3 = pl.pallas_call(
        _mlp_block,
        grid=(e,),
        in_specs=[
            pl.BlockSpec((rows, h), lambda ei: (0, ei)),
            pl.BlockSpec((1, h, i), lambda ei: (ei, 0, 0)),
            pl.BlockSpec((1, h, i), lambda ei: (ei, 0, 1)),
            pl.BlockSpec((1, i // 2, h), lambda ei: (ei, 0, 0)),
            pl.BlockSpec((1, i // 2, h), lambda ei: (ei, 1, 0)),
        ],
        out_specs=pl.BlockSpec((rows, h), lambda ei: (0, ei)),
        out_shape=jax.ShapeDtypeStruct((rows, e * h), jnp.float32),
    )(x3, gate_up_proj, gate_up_proj, down_proj, down_proj)
    return out3.reshape(b, n, h)


# speedup vs baseline: 1.0058x; 1.0058x over previous
import jax
import jax.numpy as jnp
from jax.experimental import pallas as pl
from jax.experimental.pallas import tpu as pltpu


def _mlp_block(x_ref, gua_ref, gub_ref, dpa_ref, dpb_ref, o_ref):
    ih = dpa_ref.shape[1]
    xb = x_ref[...].astype(jnp.bfloat16)
    gate = jnp.dot(xb, gua_ref[0].astype(jnp.bfloat16),
                   preferred_element_type=jnp.float32)
    up = jnp.dot(xb, gub_ref[0].astype(jnp.bfloat16),
                 preferred_element_type=jnp.float32)
    inter = (jax.nn.silu(gate) * up).astype(jnp.bfloat16)
    o_ref[...] = (
        jnp.dot(inter[:, :ih], dpa_ref[0].astype(jnp.bfloat16),
                preferred_element_type=jnp.float32)
        + jnp.dot(inter[:, ih:], dpb_ref[0].astype(jnp.bfloat16),
                  preferred_element_type=jnp.float32)
    )


def kernel(x, gate_up_proj, down_proj):
    b, n, h = x.shape
    e, _, i2 = gate_up_proj.shape
    i = i2 // 2
    g = n // e
    rows = b * g
    x3 = x.reshape(rows, e * h)
    out3 = pl.pallas_call(
        _mlp_block,
        grid=(e,),
        in_specs=[
            pl.BlockSpec((rows, h), lambda ei: (0, ei)),
            pl.BlockSpec((1, h, i), lambda ei: (ei, 0, 0)),
            pl.BlockSpec((1, h, i), lambda ei: (ei, 0, 1)),
            pl.BlockSpec((1, i // 2, h), lambda ei: (ei, 0, 0)),
            pl.BlockSpec((1, i // 2, h), lambda ei: (ei, 1, 0)),
        ],
        out_specs=pl.BlockSpec((rows, h), lambda ei: (0, ei)),
        out_shape=jax.ShapeDtypeStruct((rows, e * h), jnp.float32),
        compiler_params=pltpu.CompilerParams(
            dimension_semantics=("parallel",)),
    )(x3, gate_up_proj, gate_up_proj, down_proj, down_proj)
    return out3.reshape(b, n, h)


# speedup vs baseline: 1.2725x; 1.2652x over previous
import jax
import jax.numpy as jnp
from jax.experimental import pallas as pl


def _mlp_step(x_ref, gup_ref, dp_ref, o_ref):
    ei = pl.program_id(1)
    ih = dp_ref.shape[1]
    xe = x_ref[0, :, ei, :].astype(jnp.bfloat16)
    gu = jnp.dot(xe, gup_ref[0].astype(jnp.bfloat16),
                 preferred_element_type=jnp.float32)
    inter = (jax.nn.silu(gu[:, :ih]) * gu[:, ih:]).astype(jnp.bfloat16)
    o_ref[0, :, ei, :] = jnp.dot(inter, dp_ref[0].astype(jnp.bfloat16),
                                 preferred_element_type=jnp.float32)


def kernel(x, gate_up_proj, down_proj):
    b, n, h = x.shape
    e, _, i2 = gate_up_proj.shape
    i = i2 // 2
    g = n // e
    x4 = x.reshape(b, g, e, h)
    out4 = pl.pallas_call(
        _mlp_step,
        grid=(b, e),
        in_specs=[
            pl.BlockSpec((1, g, e, h), lambda bi, ei: (bi, 0, 0, 0)),
            pl.BlockSpec((1, h, i2), lambda bi, ei: (ei, 0, 0)),
            pl.BlockSpec((1, i, h), lambda bi, ei: (ei, 0, 0)),
        ],
        out_specs=pl.BlockSpec((1, g, e, h), lambda bi, ei: (bi, 0, 0, 0)),
        out_shape=jax.ShapeDtypeStruct((b, g, e, h), jnp.float32),
    )(x4, gate_up_proj, down_proj)
    return out4.reshape(b, n, h)


# speedup vs baseline: 1.5763x; 1.2387x over previous
import jax
import jax.numpy as jnp
from jax.experimental import pallas as pl
from jax.experimental.pallas import tpu as pltpu


def _mlp_step(x_hbm, gup_ref, dp_ref, o_hbm, xbuf, obuf, lsem, ssem):
    ei = pl.program_id(0)
    bi = pl.program_id(1)
    nb = pl.num_programs(1)
    nsteps = pl.num_programs(0) * nb
    k = ei * nb + bi
    slot = jax.lax.rem(k, 2)
    nslot = jax.lax.rem(k + 1, 2)

    def load(kk, sl):
        return pltpu.make_async_copy(
            x_hbm.at[jax.lax.rem(kk, nb), :, kk // nb, :],
            xbuf.at[sl], lsem.at[sl])

    @pl.when(k == 0)
    def _():
        load(k, slot).start()

    @pl.when(k + 1 < nsteps)
    def _():
        load(k + 1, nslot).start()

    load(k, slot).wait()

    ih = dp_ref.shape[1]
    xe = xbuf[slot].astype(jnp.bfloat16)
    gu = jnp.dot(xe, gup_ref[0].astype(jnp.bfloat16),
                 preferred_element_type=jnp.float32)
    inter = (jax.nn.silu(gu[:, :ih]) * gu[:, ih:]).astype(jnp.bfloat16)

    def store(sl):
        return pltpu.make_async_copy(
            obuf.at[sl], o_hbm.at[bi, :, ei, :], ssem.at[sl])

    @pl.when(k >= 2)
    def _():
        store(slot).wait()

    obuf[slot] = jnp.dot(inter, dp_ref[0].astype(jnp.bfloat16),
                         preferred_element_type=jnp.float32)
    store(slot).start()

    @pl.when(k == nsteps - 1)
    def _():
        store(slot).wait()

        @pl.when(nsteps >= 2)
        def _():
            store(nslot).wait()


def kernel(x, gate_up_proj, down_proj):
    b, n, h = x.shape
    e, _, i2 = gate_up_proj.shape
    i = i2 // 2
    g = n // e
    x4 = x.reshape(b, g, e, h)
    out4 = pl.pallas_call(
        _mlp_step,
        grid=(e, b),
        in_specs=[
            pl.BlockSpec(memory_space=pl.ANY),
            pl.BlockSpec((1, h, i2), lambda ei, bi: (ei, 0, 0)),
            pl.BlockSpec((1, i, h), lambda ei, bi: (ei, 0, 0)),
        ],
        out_specs=pl.BlockSpec(memory_space=pl.ANY),
        out_shape=jax.ShapeDtypeStruct((b, g, e, h), jnp.float32),
        scratch_shapes=[
            pltpu.VMEM((2, g, h), jnp.float32),
            pltpu.VMEM((2, g, h), jnp.float32),
            pltpu.SemaphoreType.DMA((2,)),
            pltpu.SemaphoreType.DMA((2,)),
        ],
    )(x4, gate_up_proj, down_proj)
    return out4.reshape(b, n, h)


# speedup vs baseline: 2.2501x; 1.4275x over previous
import jax
import jax.numpy as jnp
from jax.experimental import pallas as pl
from jax.experimental.pallas import tpu as pltpu


def _mlp_step(x_hbm, gup_hbm, dp_hbm, o_hbm,
              xbuf, obuf, wgu_stage, wdp_stage, wgu16, wdp16,
              lsem, ssem, wgsem, wdsem):
    ei = pl.program_id(0)
    ne = pl.num_programs(0)
    slot = jax.lax.rem(ei, 2)
    nslot = jax.lax.rem(ei + 1, 2)

    def xload(kk, sl):
        return pltpu.make_async_copy(
            x_hbm.at[:, :, kk, :], xbuf.at[sl], lsem.at[sl])

    def wguload(kk, sl):
        return pltpu.make_async_copy(
            gup_hbm.at[kk], wgu_stage.at[sl], wgsem.at[sl])

    def wdpload(kk, sl):
        return pltpu.make_async_copy(
            dp_hbm.at[kk], wdp_stage.at[sl], wdsem.at[sl])

    @pl.when(ei == 0)
    def _():
        xload(ei, slot).start()
        wguload(ei, slot).start()
        wdpload(ei, slot).start()

    @pl.when(ei + 1 < ne)
    def _():
        xload(ei + 1, nslot).start()
        wguload(ei + 1, nslot).start()
        wdpload(ei + 1, nslot).start()

    wguload(ei, slot).wait()
    wdpload(ei, slot).wait()
    wgu16[...] = wgu_stage[slot].astype(jnp.bfloat16)
    wdp16[...] = wdp_stage[slot].astype(jnp.bfloat16)

    xload(ei, slot).wait()

    bb, gg, hh = xbuf.shape[1], xbuf.shape[2], xbuf.shape[3]
    ih = wdp16.shape[0]
    xe = xbuf[slot].reshape(bb * gg, hh).astype(jnp.bfloat16)
    gu = jnp.dot(xe, wgu16[...], preferred_element_type=jnp.float32)
    inter = (jax.nn.silu(gu[:, :ih]) * gu[:, ih:]).astype(jnp.bfloat16)

    def store(sl):
        return pltpu.make_async_copy(
            obuf.at[sl], o_hbm.at[:, :, ei, :], ssem.at[sl])

    @pl.when(ei >= 2)
    def _():
        store(slot).wait()

    obuf[slot] = jnp.dot(inter, wdp16[...],
                         preferred_element_type=jnp.float32).reshape(bb, gg, hh)
    store(slot).start()

    @pl.when(ei == ne - 1)
    def _():
        store(slot).wait()
        store(nslot).wait()


def kernel(x, gate_up_proj, down_proj):
    b, n, h = x.shape
    e, _, i2 = gate_up_proj.shape
    i = i2 // 2
    g = n // e
    x4 = x.reshape(b, g, e, h)
    out4 = pl.pallas_call(
        _mlp_step,
        grid=(e,),
        in_specs=[
            pl.BlockSpec(memory_space=pl.ANY),
            pl.BlockSpec(memory_space=pl.ANY),
            pl.BlockSpec(memory_space=pl.ANY),
        ],
        out_specs=pl.BlockSpec(memory_space=pl.ANY),
        out_shape=jax.ShapeDtypeStruct((b, g, e, h), jnp.float32),
        scratch_shapes=[
            pltpu.VMEM((2, b, g, h), jnp.float32),
            pltpu.VMEM((2, b, g, h), jnp.float32),
            pltpu.VMEM((2, h, i2), jnp.float32),
            pltpu.VMEM((2, i, h), jnp.float32),
            pltpu.VMEM((h, i2), jnp.bfloat16),
            pltpu.VMEM((i, h), jnp.bfloat16),
            pltpu.SemaphoreType.DMA((2,)),
            pltpu.SemaphoreType.DMA((2,)),
            pltpu.SemaphoreType.DMA((2,)),
            pltpu.SemaphoreType.DMA((2,)),
        ],
    )(x4, gate_up_proj, down_proj)
    return out4.reshape(b, n, h)
